# Initial kernel scaffold; baseline (speedup 1.0000x reference)
import jax
import jax.numpy as jnp
from jax.experimental import pallas as pl


def kernel(logits, top_k):
    raise NotImplementedError("write your pallas kernel here")


# speedup vs baseline: 2.7555x; 2.7555x over previous
import jax
import jax.numpy as jnp
import numpy as np
from jax import lax
from jax.experimental import pallas as pl
from jax.experimental.pallas import tpu as pltpu
from jax.experimental.pallas import tpu_sc as plsc

B = 128
V = 100000
K = 64
L = 16
NC = 2
NS = 16
NW = NC * NS
RPW = B // NW
CHUNK = 20000
NCHUNK = V // CHUNK
GROUP = 160
NVPG = GROUP // L
NGROUP = CHUNK // GROUP
CAP = 480
CAPP = CAP + L
NCV = CAPP // L
NEG = np.float32(-np.inf)


def _splat_f(x):
    return jnp.broadcast_to(x, (L,)).astype(jnp.float32)


def _splat_i(x):
    return jnp.broadcast_to(x, (L,)).astype(jnp.int32)


def _sget(ref, i):
    return jnp.max(plsc.load_gather(ref, [_splat_i(i)]))


def _sput(ref, i, x, lane):
    plsc.store_scatter(ref, [_splat_i(i)], jnp.broadcast_to(x, (L,)),
                       mask=lane == 0)


def _body(logits_hbm, val_hbm, idx_hbm,
          buf0, buf1, cand_val, cand_idx, top_val, top_idx, sem0, sem1):
    cid = lax.axis_index("c")
    sid = lax.axis_index("s")
    wid = sid * NC + cid
    bufs = (buf0, buf1)
    sems = (sem0, sem1)
    lane = lax.iota(jnp.int32, L)
    zero16_i = jnp.zeros((L,), jnp.int32)
    neg16 = jnp.full((L,), NEG, jnp.float32)

    def clear_cand():
        for i in range(NCV):
            cand_val[pl.ds(i * L, L)] = neg16
            cand_idx[pl.ds(i * L, L)] = _splat_i(V)

    def select_topk():
        def ext_step(t, _):
            m = cand_val[pl.ds(0, L)]
            cidv = zero16_i
            for i in range(1, NCV):
                vv = cand_val[pl.ds(i * L, L)]
                upd = vv > m
                m = jnp.where(upd, vv, m)
                cidv = jnp.where(upd, jnp.full((L,), i, jnp.int32), cidv)
            maxv = jnp.max(m)
            posv = jnp.where(m == maxv, cidv * L + lane,
                             jnp.full((L,), 1 << 20, jnp.int32))
            pos = jnp.min(posv)
            idx_w = _sget(cand_idx, pos)
            _sput(top_val, t, maxv, lane)
            _sput(top_idx, t, idx_w, lane)
            _sput(cand_val, pos, NEG, lane)
            return 0
        lax.fori_loop(0, K, ext_step, 0)

    def compact(cur_theta):
        select_topk()
        clear_cand()
        for i in range(K // L):
            cand_val[pl.ds(i * L, L)] = top_val[pl.ds(i * L, L)]
            cand_idx[pl.ds(i * L, L)] = top_idx[pl.ds(i * L, L)]
        theta_v = plsc.load_gather(top_val, [_splat_i(K - 1)])
        return _splat_i(K), theta_v

    def row_body(r, _):
        row = wid * RPW + r
        base = row * V
        clear_cand()
        cps = [None] * NCHUNK
        cps[0] = pltpu.async_copy(
            logits_hbm.at[pl.ds(base, CHUNK)], bufs[0], sems[0])
        cur = jnp.zeros((L,), jnp.int32)
        theta = neg16
        lane15 = jnp.full((L,), L - 1, jnp.int32)
        for c in range(NCHUNK):
            cps[c].wait()
            if c + 1 < NCHUNK:
                cps[c + 1] = pltpu.async_copy(
                    logits_hbm.at[pl.ds(base + (c + 1) * CHUNK, CHUNK)],
                    bufs[(c + 1) % 2], sems[(c + 1) % 2])
            buf = bufs[c % 2]
            col0 = c * CHUNK

            def group_body(gi, carry, buf=buf, col0=col0):
                cur, theta = carry
                off = gi * GROUP
                for v in range(NVPG):
                    x = buf[pl.ds(off + v * L, L)]
                    m = x > theta
                    ps = plsc.cumsum(m.astype(jnp.int32))
                    pos = cur + ps - 1
                    colv = _splat_i(col0 + v * L) + _splat_i(off) + lane
                    plsc.store_scatter(cand_val, [pos], x, mask=m)
                    plsc.store_scatter(cand_idx, [pos], colv, mask=m)
                    cur = cur + jnp.take_along_axis(ps, lane15, axis=0)
                cur_s = jnp.max(cur)
                cur, theta = lax.cond(cur_s >= CAP - GROUP, compact,
                                      lambda ct: ct, (cur, theta))
                return cur, theta

            cur, theta = lax.fori_loop(0, NGROUP, group_body, (cur, theta))

        select_topk()
        pltpu.sync_copy(top_val, val_hbm.at[row])
        pltpu.sync_copy(top_idx, idx_hbm.at[row])
        return 0

    lax.fori_loop(0, RPW, row_body, 0)


def kernel(logits, top_k):
    kfn = pl.kernel(
        _body,
        out_type=(jax.ShapeDtypeStruct((B, K), jnp.float32),
                  jax.ShapeDtypeStruct((B, K), jnp.int32)),
        mesh=plsc.VectorSubcoreMesh(core_axis_name="c", subcore_axis_name="s"),
        compiler_params=pltpu.CompilerParams(needs_layout_passes=False),
        scratch_types=[
            pltpu.VMEM((CHUNK,), jnp.float32),
            pltpu.VMEM((CHUNK,), jnp.float32),
            pltpu.VMEM((CAPP,), jnp.float32),
            pltpu.VMEM((CAPP,), jnp.int32),
            pltpu.VMEM((K,), jnp.float32),
            pltpu.VMEM((K,), jnp.int32),
            pltpu.SemaphoreType.DMA,
            pltpu.SemaphoreType.DMA,
        ],
    )
    logits_top, indices = kfn(logits.reshape(B * V))

    logits_top = (logits_top + 0 * top_k) / 0.8
    sorted_logits = jnp.sort(logits_top, axis=-1)
    sorted_idx = jnp.argsort(logits_top, axis=-1)
    cumulative_probs = jnp.cumsum(jax.nn.softmax(sorted_logits, axis=-1),
                                  axis=-1)
    sorted_to_remove = cumulative_probs <= (1.0 - 0.9)
    remove = jnp.zeros((B, K), dtype=bool)
    remove = remove.at[jnp.arange(B)[:, None], sorted_idx].set(
        sorted_to_remove)
    filtered = jnp.where(remove, -jnp.inf, logits_top)
    skey = jax.random.fold_in(jax.random.key(0), 1)
    choice = jax.random.categorical(skey, filtered, axis=-1)
    return indices[jnp.arange(B), choice]
